# baseline (speedup 1.0000x reference)
import dataclasses
import functools

import jax
import jax.numpy as jnp
from jax.experimental import pallas as pl
from jax.experimental.pallas import tpu as pltpu
from jax.experimental.pallas import tpu_sc as plsc

N, D = 16384, 128
L = 16
TILE_R = 64
TILE_E = TILE_R * D
N_SC = 2048
TC_BLOCK_R = 8192
LD_BLOCK = TC_BLOCK_R // D

_mesh = plsc.VectorSubcoreMesh(core_axis_name="c", subcore_axis_name="s")

_cp = pltpu.CompilerParams()
if "needs_layout_passes" in pltpu.CompilerParams.__dataclass_fields__:
    _cp = dataclasses.replace(_cp, needs_layout_passes=False)


@functools.partial(
    pl.kernel,
    out_type=jax.ShapeDtypeStruct((N_SC,), jnp.float32),
    mesh=_mesh,
    compiler_params=_cp,
    scratch_types=[pltpu.VMEM((L,), jnp.float32)],
)
def _sc_logabsdet(ctx_hbm, lv_hbm, ld_hbm, lv_v):
    pltpu.sync_copy(lv_hbm, lv_v)
    zero = jnp.zeros((L,), jnp.float32)
    lane = jnp.arange(L, dtype=jnp.int32)

    def body(ctx_t, ld_t):
        lv = lv_v[...]
        for g in range(TILE_R // L):
            merged = zero
            for j in range(L):
                acc = zero
                for c in range(D // L):
                    off = (g * L + j) * D + c * L
                    t = ctx_t[pl.ds(off, L)]
                    acc = acc + jnp.where(t > 0.0, lv, zero)
                merged = jnp.where(lane == j, jnp.sum(acc), merged)
            ld_t[pl.ds(g * L, L)] = merged

    pltpu.emit_pipeline(
        body,
        grid=(N_SC // TILE_R,),
        in_specs=[pl.BlockSpec((TILE_E,), lambda i: (i,))],
        out_specs=[pl.BlockSpec((TILE_R,), lambda i: (i,))],
        core_axis_name=("c", "s"),
        dimension_semantics=(pltpu.PARALLEL,),
    )(ctx_hbm, ld_hbm)


def _tc_body(x_ref, c_ref, s_ref, b_ref, lv_ref, o_ref, ld_ref):
    c = c_ref[...]
    mask = c > 0.0
    o_ref[...] = jnp.where(mask, x_ref[...] * s_ref[0, 0] + b_ref[0, 0],
                           x_ref[...])
    ones = jnp.full((D, 1), 1.0, dtype=jnp.float32)
    counts = jax.lax.dot_general(
        mask.astype(jnp.float32), ones,
        (((1,), (0,)), ((), ())),
        preferred_element_type=jnp.float32)
    ld_ref[...] = counts.reshape(LD_BLOCK, D) * lv_ref[0, 0]


_tc_transform = pl.pallas_call(
    _tc_body,
    grid=(N // TC_BLOCK_R,),
    in_specs=[
        pl.BlockSpec((TC_BLOCK_R, D), lambda i: (i, 0)),
        pl.BlockSpec((TC_BLOCK_R, D), lambda i: (i, 0)),
        pl.BlockSpec((1, 1), lambda i: (0, 0)),
        pl.BlockSpec((1, 1), lambda i: (0, 0)),
        pl.BlockSpec((1, 1), lambda i: (0, 0)),
    ],
    out_specs=[
        pl.BlockSpec((TC_BLOCK_R, D), lambda i: (i, 0)),
        pl.BlockSpec((LD_BLOCK, D), lambda i: (i, 0)),
    ],
    out_shape=[
        jax.ShapeDtypeStruct((N, D), jnp.float32),
        jax.ShapeDtypeStruct((N // D, D), jnp.float32),
    ],
)


def kernel(inputs, context, log_scale, shift):
    lv = jnp.broadcast_to(log_scale, (L,))
    ld_sc = _sc_logabsdet(context[:N_SC].reshape(N_SC * D), lv)
    sv = jnp.exp(log_scale).reshape(1, 1)
    bv = shift.reshape(1, 1)
    lvs = log_scale.reshape(1, 1)
    outputs, ld = _tc_transform(inputs, context, sv, bv, lvs)
    logabsdet = jnp.concatenate([ld_sc, ld.reshape(N)[N_SC:]])
    return outputs, logabsdet


# speedup vs baseline: 2.9679x; 2.9679x over previous
import jax
import jax.numpy as jnp
from jax.experimental import pallas as pl

N, D = 16384, 128
TC_BLOCK_R = 8192
LD_BLOCK = TC_BLOCK_R // D


def _tc_body(x_ref, c_ref, lv_ref, b_ref, o_ref, ld_ref):
    c = c_ref[...]
    mask = c > 0.0
    s = jnp.exp(lv_ref[0, 0])
    o_ref[...] = jnp.where(mask, x_ref[...] * s + b_ref[0, 0],
                           x_ref[...])
    ones = jnp.full((D, 1), 1.0, dtype=jnp.float32)
    counts = jax.lax.dot_general(
        mask.astype(jnp.float32), ones,
        (((1,), (0,)), ((), ())),
        preferred_element_type=jnp.float32)
    ld_ref[...] = counts.reshape(LD_BLOCK, D) * lv_ref[0, 0]


_tc_transform = pl.pallas_call(
    _tc_body,
    grid=(N // TC_BLOCK_R,),
    in_specs=[
        pl.BlockSpec((TC_BLOCK_R, D), lambda i: (i, 0)),
        pl.BlockSpec((TC_BLOCK_R, D), lambda i: (i, 0)),
        pl.BlockSpec((1, 1), lambda i: (0, 0)),
        pl.BlockSpec((1, 1), lambda i: (0, 0)),
    ],
    out_specs=[
        pl.BlockSpec((TC_BLOCK_R, D), lambda i: (i, 0)),
        pl.BlockSpec((LD_BLOCK, D), lambda i: (i, 0)),
    ],
    out_shape=[
        jax.ShapeDtypeStruct((N, D), jnp.float32),
        jax.ShapeDtypeStruct((N // D, D), jnp.float32),
    ],
)


def kernel(inputs, context, log_scale, shift):
    lvs = log_scale.reshape(1, 1)
    bv = shift.reshape(1, 1)
    outputs, ld = _tc_transform(inputs, context, lvs, bv)
    return outputs, ld.reshape(N)
